# Initial kernel scaffold; baseline (speedup 1.0000x reference)
import jax
import jax.numpy as jnp
from jax.experimental import pallas as pl


def kernel(x, indmap, embmap, embs, W1, b1, scales, missing_w):
    raise NotImplementedError("write your pallas kernel here")


# speedup vs baseline: 1.2087x; 1.2087x over previous
import functools

import jax
import jax.numpy as jnp
from jax import lax
from jax.experimental import pallas as pl
from jax.experimental.pallas import tpu as pltpu
from jax.experimental.pallas import tpu_sc as plsc

N_CLASSES = 100000
N_EMBS = 90000
TEXT_DIM = 768
EMB_DIM = 64
BATCH = 16384

NC, NS, L = 2, 16, 16
NW = NC * NS
B_PER_W = BATCH // NW
CH = 64
NCH = B_PER_W // CH


def _sc_mesh():
    return plsc.VectorSubcoreMesh(
        core_axis_name="c", subcore_axis_name="s", num_cores=NC, num_subcores=NS
    )


@functools.partial(
    pl.kernel,
    mesh=_sc_mesh(),
    out_type=[
        jax.ShapeDtypeStruct((BATCH, TEXT_DIM), jnp.float32),
        jax.ShapeDtypeStruct((BATCH, 128), jnp.float32),
        jax.ShapeDtypeStruct((NW, NCH, CH), jnp.float32),
        jax.ShapeDtypeStruct((NW, NCH, CH), jnp.int32),
    ],
    scratch_types=[
        pltpu.VMEM((NCH, CH), jnp.int32),
        pltpu.VMEM((NCH, CH), jnp.int32),
        pltpu.VMEM((NCH, CH), jnp.int32),
        pltpu.VMEM((NCH, CH), jnp.int32),
        pltpu.VMEM((NCH, CH), jnp.float32),
        pltpu.VMEM((CH, TEXT_DIM), jnp.float32),
        pltpu.VMEM((CH, TEXT_DIM), jnp.float32),
        pltpu.VMEM((CH, 128), jnp.float32),
        pltpu.SemaphoreType.DMA,
        pltpu.SemaphoreType.DMA,
    ],
)
def _sc_gather(x_hbm, indmap_hbm, embmap_hbm, scales_hbm, embs_hbm, missing_hbm,
               rows_out, repl_out, scl_out, inds_out,
               xv, indv, siv, emv, sclv, rows_a, rows_b, replv, sem_g, sem_w):
    wid = lax.axis_index("s") * NC + lax.axis_index("c")
    base = wid * B_PER_W

    pltpu.sync_copy(x_hbm.at[wid], xv)
    for j in range(NCH):
        pltpu.async_copy(indmap_hbm.at[xv.at[j]], indv.at[j], sem_g).wait()
        pltpu.async_copy(scales_hbm.at[xv.at[j]], sclv.at[j], sem_g).wait()
        pltpu.async_copy(embmap_hbm.at[xv.at[j]], emv.at[j], sem_g).wait()
    for j in range(NCH):
        for i in range(CH // L):
            sl = pl.ds(i * L, L)
            siv[j, sl] = jnp.maximum(indv[j, sl], 0)
            emv[j, sl] = jnp.maximum(emv[j, sl], 0)

    pltpu.sync_copy(indv, inds_out.at[wid])
    pltpu.sync_copy(sclv, scl_out.at[wid])

    for j in range(NCH):
        pltpu.async_copy(missing_hbm.at[emv.at[j]], replv, sem_g).wait()
        pltpu.sync_copy(replv, repl_out.at[pl.ds(base + j * CH, CH)])

    bufs = (rows_a, rows_b)
    copies = [None, None]
    for j in range(NCH):
        buf = bufs[j % 2]
        if copies[j % 2] is not None:
            copies[j % 2].wait()
        pltpu.async_copy(embs_hbm.at[siv.at[j]], buf, sem_g).wait()
        cp = pltpu.async_copy(buf, rows_out.at[pl.ds(base + j * CH, CH)], sem_w)
        copies[j % 2] = cp
    copies[0].wait()
    copies[1].wait()


_ROWS_BLK = 512


def _tc_body(rows_ref, w_ref, b_ref, scl_ref, inds_ref, repl_ref, out_ref):
    rows = rows_ref[...]
    y = lax.dot_general(
        rows, w_ref[...], (((1,), (1,)), ((), ())),
        preferred_element_type=jnp.float32,
    ) + b_ref[...]
    mean = jnp.mean(y, axis=1, keepdims=True)
    d = y - mean
    var = jnp.sum(d * d, axis=1, keepdims=True) * (1.0 / (EMB_DIM - 1))
    res = scl_ref[...] * y * lax.rsqrt(var)
    repl = repl_ref[...][:, :EMB_DIM]
    out_ref[...] = jnp.where(inds_ref[...] < 0, repl, res)


def kernel(x, indmap, embmap, embs, W1, b1, scales, missing_w):
    x3 = x.reshape(NW, NCH, CH)
    scales1 = scales.reshape(N_CLASSES)
    missing_pad = jnp.pad(missing_w, ((0, 0), (0, 128 - EMB_DIM)))
    rows, repl, scl, inds = _sc_gather(
        x3, indmap, embmap, scales1, embs, missing_pad
    )
    scl2 = scl.reshape(BATCH, 1)
    inds2 = inds.reshape(BATCH, 1)
    b2 = b1.reshape(1, EMB_DIM)

    grid = (BATCH // _ROWS_BLK,)
    out = pl.pallas_call(
        _tc_body,
        grid=grid,
        in_specs=[
            pl.BlockSpec((_ROWS_BLK, TEXT_DIM), lambda i: (i, 0)),
            pl.BlockSpec((EMB_DIM, TEXT_DIM), lambda i: (0, 0)),
            pl.BlockSpec((1, EMB_DIM), lambda i: (0, 0)),
            pl.BlockSpec((_ROWS_BLK, 1), lambda i: (i, 0)),
            pl.BlockSpec((_ROWS_BLK, 1), lambda i: (i, 0)),
            pl.BlockSpec((_ROWS_BLK, 128), lambda i: (i, 0)),
        ],
        out_specs=pl.BlockSpec((_ROWS_BLK, EMB_DIM), lambda i: (i, 0)),
        out_shape=jax.ShapeDtypeStruct((BATCH, EMB_DIM), jnp.float32),
    )(rows, W1, b2, scl2, inds2, repl)
    return out


# speedup vs baseline: 1.3025x; 1.0776x over previous
import functools

import jax
import jax.numpy as jnp
from jax import lax
from jax.experimental import pallas as pl
from jax.experimental.pallas import tpu as pltpu
from jax.experimental.pallas import tpu_sc as plsc

N_CLASSES = 100000
N_EMBS = 90000
TEXT_DIM = 768
EMB_DIM = 64
BATCH = 16384

NC, NS, L = 2, 16, 16
NW = NC * NS
B_PER_W = BATCH // NW
CH = 64
NCH = B_PER_W // CH


def _sc_mesh():
    return plsc.VectorSubcoreMesh(
        core_axis_name="c", subcore_axis_name="s", num_cores=NC, num_subcores=NS
    )


@functools.partial(
    pl.kernel,
    mesh=_sc_mesh(),
    out_type=[
        jax.ShapeDtypeStruct((BATCH, TEXT_DIM), jnp.float32),
        jax.ShapeDtypeStruct((BATCH, 128), jnp.float32),
        jax.ShapeDtypeStruct((NW, NCH, CH), jnp.float32),
        jax.ShapeDtypeStruct((NW, NCH, CH), jnp.int32),
    ],
    scratch_types=[
        pltpu.VMEM((NCH, CH), jnp.int32),
        pltpu.VMEM((NCH, CH), jnp.int32),
        pltpu.VMEM((NCH, CH), jnp.int32),
        pltpu.VMEM((NCH, CH), jnp.int32),
        pltpu.VMEM((NCH, CH), jnp.float32),
        pltpu.VMEM((CH, TEXT_DIM), jnp.float32),
        pltpu.VMEM((CH, TEXT_DIM), jnp.float32),
        pltpu.VMEM((CH, 128), jnp.float32),
        pltpu.VMEM((CH, 128), jnp.float32),
        pltpu.SemaphoreType.DMA,
        pltpu.SemaphoreType.DMA,
        pltpu.SemaphoreType.DMA,
        pltpu.SemaphoreType.DMA,
        pltpu.SemaphoreType.DMA,
        pltpu.SemaphoreType.DMA,
        pltpu.SemaphoreType.DMA,
        pltpu.SemaphoreType.DMA,
        pltpu.SemaphoreType.DMA,
        pltpu.SemaphoreType.DMA,
    ],
)
def _sc_gather(x_hbm, indmap_hbm, embmap_hbm, scales_hbm, embs_hbm, missing_hbm,
               rows_out, repl_out, scl_out, inds_out,
               xv, indv, siv, emv, sclv, rows_a, rows_b, repl_a, repl_b,
               sem_g, sem_o, sgr_a, sgr_b, swr_a, swr_b, sgm_a, sgm_b,
               swm_a, swm_b):
    wid = lax.axis_index("s") * NC + lax.axis_index("c")
    base = wid * B_PER_W

    pltpu.sync_copy(x_hbm.at[wid], xv)
    descs = []
    for j in range(NCH):
        descs.append(pltpu.async_copy(indmap_hbm.at[xv.at[j]], indv.at[j], sem_g))
        descs.append(pltpu.async_copy(scales_hbm.at[xv.at[j]], sclv.at[j], sem_g))
        descs.append(pltpu.async_copy(embmap_hbm.at[xv.at[j]], emv.at[j], sem_g))
    for d in descs:
        d.wait()
    for j in range(NCH):
        for i in range(CH // L):
            sl = pl.ds(i * L, L)
            siv[j, sl] = jnp.maximum(indv[j, sl], 0)
            emv[j, sl] = jnp.maximum(emv[j, sl], 0)

    d_inds = pltpu.async_copy(indv, inds_out.at[wid], sem_o)
    d_scl = pltpu.async_copy(sclv, scl_out.at[wid], sem_o)

    rbuf = (rows_a, rows_b)
    mbuf = (repl_a, repl_b)
    sgr = (sgr_a, sgr_b)
    swr = (swr_a, swr_b)
    sgm = (sgm_a, sgm_b)
    swm = (swm_a, swm_b)
    gr = [None, None]
    gm = [None, None]
    wr = [None, None]
    wm = [None, None]

    def issue(j):
        b = j % 2
        gr[b] = pltpu.async_copy(embs_hbm.at[siv.at[j]], rbuf[b], sgr[b])
        gm[b] = pltpu.async_copy(missing_hbm.at[emv.at[j]], mbuf[b], sgm[b])

    issue(0)
    for j in range(NCH):
        b = j % 2
        if j + 1 < NCH:
            nb = (j + 1) % 2
            if wr[nb] is not None:
                wr[nb].wait()
                wm[nb].wait()
            issue(j + 1)
        gr[b].wait()
        gm[b].wait()
        sl = pl.ds(base + j * CH, CH)
        wr[b] = pltpu.async_copy(rbuf[b], rows_out.at[sl], swr[b])
        wm[b] = pltpu.async_copy(mbuf[b], repl_out.at[sl], swm[b])
    wr[0].wait()
    wr[1].wait()
    wm[0].wait()
    wm[1].wait()
    d_inds.wait()
    d_scl.wait()


_ROWS_BLK = 512


def _tc_body(rows_ref, w_ref, b_ref, scl_ref, inds_ref, repl_ref, out_ref):
    rows = rows_ref[...]
    y = lax.dot_general(
        rows, w_ref[...], (((1,), (1,)), ((), ())),
        preferred_element_type=jnp.float32,
    ) + b_ref[...]
    mean = jnp.mean(y, axis=1, keepdims=True)
    d = y - mean
    var = jnp.sum(d * d, axis=1, keepdims=True) * (1.0 / (EMB_DIM - 1))
    res = scl_ref[...] * y * lax.rsqrt(var)
    repl = repl_ref[...][:, :EMB_DIM]
    out_ref[...] = jnp.where(inds_ref[...] < 0, repl, res)


def kernel(x, indmap, embmap, embs, W1, b1, scales, missing_w):
    x3 = x.reshape(NW, NCH, CH)
    scales1 = scales.reshape(N_CLASSES)
    missing_pad = jnp.pad(missing_w, ((0, 0), (0, 128 - EMB_DIM)))
    rows, repl, scl, inds = _sc_gather(
        x3, indmap, embmap, scales1, embs, missing_pad
    )
    scl2 = scl.reshape(BATCH, 1)
    inds2 = inds.reshape(BATCH, 1)
    b2 = b1.reshape(1, EMB_DIM)

    grid = (BATCH // _ROWS_BLK,)
    out = pl.pallas_call(
        _tc_body,
        grid=grid,
        in_specs=[
            pl.BlockSpec((_ROWS_BLK, TEXT_DIM), lambda i: (i, 0)),
            pl.BlockSpec((EMB_DIM, TEXT_DIM), lambda i: (0, 0)),
            pl.BlockSpec((1, EMB_DIM), lambda i: (0, 0)),
            pl.BlockSpec((_ROWS_BLK, 1), lambda i: (i, 0)),
            pl.BlockSpec((_ROWS_BLK, 1), lambda i: (i, 0)),
            pl.BlockSpec((_ROWS_BLK, 128), lambda i: (i, 0)),
        ],
        out_specs=pl.BlockSpec((_ROWS_BLK, EMB_DIM), lambda i: (i, 0)),
        out_shape=jax.ShapeDtypeStruct((BATCH, EMB_DIM), jnp.float32),
    )(rows, W1, b2, scl2, inds2, repl)
    return out


# speedup vs baseline: 3.8894x; 2.9862x over previous
import functools

import jax
import jax.numpy as jnp
from jax import lax
from jax.experimental import pallas as pl
from jax.experimental.pallas import tpu as pltpu
from jax.experimental.pallas import tpu_sc as plsc

N_CLASSES = 100000
N_EMBS = 90000
TEXT_DIM = 768
EMB_DIM = 64
BATCH = 16384
CW = 128

NC, NS, L = 2, 16, 16
NW = NC * NS
B_PER_W = BATCH // NW
CH = 64
NCH = B_PER_W // CH

PROJ_BLK = 720
MISS_BLK = 1000


def _proj_body(emb_ref, w_ref, b_ref, out_ref):
    y = lax.dot_general(
        emb_ref[...], w_ref[...], (((1,), (1,)), ((), ())),
        preferred_element_type=jnp.float32,
    ) + b_ref[...]
    mean = jnp.mean(y, axis=1, keepdims=True)
    d = y - mean
    var = jnp.sum(d * d, axis=1, keepdims=True) * (1.0 / (EMB_DIM - 1))
    z = y * lax.rsqrt(var)
    out_ref[...] = jnp.concatenate(
        [z, jnp.zeros((PROJ_BLK, CW - EMB_DIM), jnp.float32)], axis=1
    )


def _miss_body(miss_ref, c_ref, out_ref):
    del c_ref
    out_ref[...] = jnp.concatenate(
        [miss_ref[...], jnp.zeros((MISS_BLK, CW - EMB_DIM), jnp.float32)],
        axis=1,
    )


def _sc_mesh():
    return plsc.VectorSubcoreMesh(
        core_axis_name="c", subcore_axis_name="s", num_cores=NC, num_subcores=NS
    )


@functools.partial(
    pl.kernel,
    mesh=_sc_mesh(),
    out_type=[
        jax.ShapeDtypeStruct((BATCH, CW), jnp.float32),
        jax.ShapeDtypeStruct((NW, NCH, CH), jnp.float32),
        jax.ShapeDtypeStruct((NW, NCH, CH), jnp.int32),
    ],
    scratch_types=[
        pltpu.VMEM((NCH, CH), jnp.int32),
        pltpu.VMEM((NCH, CH), jnp.int32),
        pltpu.VMEM((NCH, CH), jnp.int32),
        pltpu.VMEM((NCH, CH), jnp.float32),
        pltpu.VMEM((B_PER_W, CW), jnp.float32),
        pltpu.SemaphoreType.DMA,
        pltpu.SemaphoreType.DMA,
        pltpu.SemaphoreType.DMA,
    ],
)
def _sc_gather(x_hbm, indmap_hbm, embmap_hbm, scales_hbm, comb_hbm,
               rows_out, scl_out, inds_out,
               xv, indv, uv, sclv, gv, sem_g, sem_r, sem_w):
    wid = lax.axis_index("s") * NC + lax.axis_index("c")
    base = wid * B_PER_W

    pltpu.sync_copy(x_hbm.at[wid], xv)
    descs = []
    for j in range(NCH):
        descs.append(pltpu.async_copy(indmap_hbm.at[xv.at[j]], indv.at[j], sem_g))
        descs.append(pltpu.async_copy(scales_hbm.at[xv.at[j]], sclv.at[j], sem_g))
        descs.append(pltpu.async_copy(embmap_hbm.at[xv.at[j]], uv.at[j], sem_g))
    for d in descs:
        d.wait()

    for j in range(NCH):
        for i in range(CH // L):
            sl = pl.ds(i * L, L)
            ind = indv[j, sl]
            em = jnp.maximum(uv[j, sl], 0) + N_EMBS
            u = jnp.where(ind >= 0, ind, em)
            uv[j, sl] = jnp.minimum(jnp.maximum(u, 0), N_CLASSES - 1)

    d_inds = pltpu.async_copy(indv, inds_out.at[wid], sem_w)
    d_scl = pltpu.async_copy(sclv, scl_out.at[wid], sem_w)

    gdescs = []
    for j in range(NCH):
        gdescs.append(
            pltpu.async_copy(comb_hbm.at[uv.at[j]], gv.at[pl.ds(j * CH, CH)],
                             sem_r)
        )
    for d in gdescs:
        d.wait()
    pltpu.sync_copy(gv, rows_out.at[pl.ds(base, B_PER_W)])
    d_inds.wait()
    d_scl.wait()


_OUT_BLK = 512


def _final_body(g_ref, scl_ref, inds_ref, out_ref):
    g = g_ref[...][:, :EMB_DIM]
    out_ref[...] = jnp.where(inds_ref[...] < 0, g, scl_ref[...] * g)


def kernel(x, indmap, embmap, embs, W1, b1, scales, missing_w):
    x3 = x.reshape(NW, NCH, CH)
    scales1 = scales.reshape(N_CLASSES)
    b2 = b1.reshape(1, EMB_DIM)

    comb = pl.pallas_call(
        _proj_body,
        grid=(N_EMBS // PROJ_BLK,),
        in_specs=[
            pl.BlockSpec((PROJ_BLK, TEXT_DIM), lambda i: (i, 0)),
            pl.BlockSpec((EMB_DIM, TEXT_DIM), lambda i: (0, 0)),
            pl.BlockSpec((1, EMB_DIM), lambda i: (0, 0)),
        ],
        out_specs=pl.BlockSpec((PROJ_BLK, CW), lambda i: (i, 0)),
        out_shape=jax.ShapeDtypeStruct((N_CLASSES, CW), jnp.float32),
    )(embs, W1, b2)

    comb = pl.pallas_call(
        _miss_body,
        grid=((N_CLASSES - N_EMBS) // MISS_BLK,),
        in_specs=[
            pl.BlockSpec((MISS_BLK, EMB_DIM), lambda i: (i, 0)),
            pl.BlockSpec((8, CW), lambda i: (0, 0)),
        ],
        out_specs=pl.BlockSpec(
            (MISS_BLK, CW), lambda i: (N_EMBS // MISS_BLK + i, 0)
        ),
        out_shape=jax.ShapeDtypeStruct((N_CLASSES, CW), jnp.float32),
        input_output_aliases={1: 0},
    )(missing_w, comb)

    rows, scl, inds = _sc_gather(x3, indmap, embmap, scales1, comb)
    scl2 = scl.reshape(BATCH, 1)
    inds2 = inds.reshape(BATCH, 1)

    out = pl.pallas_call(
        _final_body,
        grid=(BATCH // _OUT_BLK,),
        in_specs=[
            pl.BlockSpec((_OUT_BLK, CW), lambda i: (i, 0)),
            pl.BlockSpec((_OUT_BLK, 1), lambda i: (i, 0)),
            pl.BlockSpec((_OUT_BLK, 1), lambda i: (i, 0)),
        ],
        out_specs=pl.BlockSpec((_OUT_BLK, EMB_DIM), lambda i: (i, 0)),
        out_shape=jax.ShapeDtypeStruct((BATCH, EMB_DIM), jnp.float32),
    )(rows, scl2, inds2)
    return out
